# Initial kernel scaffold; baseline (speedup 1.0000x reference)
import jax
import jax.numpy as jnp
from jax.experimental import pallas as pl


def kernel(target, context, W_target, W_context):
    raise NotImplementedError("write your pallas kernel here")


# speedup vs baseline: 2.8246x; 2.8246x over previous
import jax
import jax.numpy as jnp
from jax import lax
from jax.experimental import pallas as pl
from jax.experimental.pallas import tpu as pltpu
from jax.experimental.pallas import tpu_sc as plsc

VOCAB = 100000
EMBED = 128
BATCH = 4096
NUM_CTX = 5

NUM_CORES = 2
NUM_SUBCORES = 16
NUM_WORKERS = NUM_CORES * NUM_SUBCORES
B_PER_W = BATCH // NUM_WORKERS
LANES = 16
K_CHUNKS = EMBED // LANES


def _sc_kernel_body(tgt_idx_hbm, ctx_idx_hbm, w_tgt_hbm, w_ctx_hbm, out_hbm,
                    tgt_idx_v, ctx_idx_v, tgt_rows, ctx_rows, out_v, sem):
    wid = lax.axis_index("subcore") * NUM_CORES + lax.axis_index("core")
    base = wid * B_PER_W

    pltpu.sync_copy(tgt_idx_hbm.at[pl.ds(base, B_PER_W)], tgt_idx_v)
    pltpu.sync_copy(ctx_idx_hbm.at[:, pl.ds(base, B_PER_W)], ctx_idx_v)

    cps = [pltpu.async_copy(w_tgt_hbm.at[tgt_idx_v], tgt_rows, sem)]
    for c in range(NUM_CTX):
        cps.append(pltpu.async_copy(
            w_ctx_hbm.at[ctx_idx_v.at[c]],
            ctx_rows.at[pl.ds(c * B_PER_W, B_PER_W)], sem))
    for cp in cps:
        cp.wait()

    last_lane = lax.iota(jnp.int32, LANES) == (LANES - 1)

    @pl.loop(0, B_PER_W)
    def _(b):
        t_chunks = [tgt_rows[b, pl.ds(k * LANES, LANES)] for k in range(K_CHUNKS)]
        for c in range(NUM_CTX):
            acc = t_chunks[0] * ctx_rows[c * B_PER_W + b, pl.ds(0, LANES)]
            for k in range(1, K_CHUNKS):
                acc = acc + t_chunks[k] * ctx_rows[c * B_PER_W + b,
                                                   pl.ds(k * LANES, LANES)]
            s = plsc.cumsum(acc)
            idx = jnp.full((LANES,), b * NUM_CTX + c, jnp.int32)
            plsc.store_scatter(out_v, [idx], s, mask=last_lane)

    pltpu.sync_copy(out_v,
                    out_hbm.at[pl.ds(base * NUM_CTX, B_PER_W * NUM_CTX)])


def kernel(target, context, W_target, W_context):
    tgt_idx = target.reshape(BATCH).astype(jnp.int32)
    ctx_idx = context.T.astype(jnp.int32)

    mesh = plsc.VectorSubcoreMesh(core_axis_name="core",
                                  subcore_axis_name="subcore")
    sc_call = pl.kernel(
        _sc_kernel_body,
        out_type=jax.ShapeDtypeStruct((BATCH * NUM_CTX,), jnp.float32),
        mesh=mesh,
        scratch_types=[
            pltpu.VMEM((B_PER_W,), jnp.int32),
            pltpu.VMEM((NUM_CTX, B_PER_W), jnp.int32),
            pltpu.VMEM((B_PER_W, EMBED), jnp.float32),
            pltpu.VMEM((NUM_CTX * B_PER_W, EMBED), jnp.float32),
            pltpu.VMEM((B_PER_W * NUM_CTX,), jnp.float32),
            pltpu.SemaphoreType.DMA,
        ],
        compiler_params=pltpu.CompilerParams(needs_layout_passes=False),
    )
    return sc_call(tgt_idx, ctx_idx, W_target, W_context).reshape(BATCH, NUM_CTX)
